# baseline (speedup 1.0000x reference)
import functools

import jax
import jax.numpy as jnp
from jax import lax
from jax.experimental import pallas as pl
from jax.experimental.pallas import tpu as pltpu
from jax.experimental.pallas import tpu_sc as plsc

B = 4096
L = 200
EMB = 64
NUM_CORES = 2
NUM_SUBCORES = 16
NW = NUM_CORES * NUM_SUBCORES
CHUNK = 128
NB = 3
TPITCH = 129

_mesh = plsc.VectorSubcoreMesh(core_axis_name="c", subcore_axis_name="s")


VOCAB = 1_000_000
NCOLS = VOCAB // 128
VPAD = (VOCAB + 127) // 128 * 128
COLS_BASE = NCOLS // NW
COLS_REM = NCOLS % NW


@functools.partial(
    pl.kernel,
    out_type=jax.ShapeDtypeStruct((VPAD // 2, 128), jnp.float32),
    mesh=_mesh,
    scratch_types=(
        [pltpu.VMEM((EMB, 128), jnp.float32) for _ in range(2)]
        + [pltpu.VMEM((EMB, TPITCH), jnp.float32) for _ in range(2)]
        + [pltpu.SemaphoreType.DMA for _ in range(4)]
    ),
    compiler_params=pltpu.CompilerParams(use_tc_tiling_on_sc=True,
                                         needs_layout_passes=False),
)
def _prep_table(tt_hbm, tail_hbm, tp_hbm, slab0, slab1, pb0, pb1,
                i0, i1, o0, o1):
    slabs = (slab0, slab1)
    pbufs = (pb0, pb1)
    isems = (i0, i1)
    osems = (o0, o1)

    wid = lax.axis_index("s") * NUM_CORES + lax.axis_index("c")
    lo = wid * COLS_BASE + jnp.minimum(wid, COLS_REM)
    extra = wid < COLS_REM

    _eg = [lax.iota(jnp.int32, 16) + 16 * g for g in range(EMB // 16)]

    def fire_in(c, k):
        off = pl.multiple_of(c * 128, 128)
        pltpu.async_copy(tt_hbm.at[:, pl.ds(off, 128)], slabs[k], isems[k])

    def drain_in(k):
        pltpu.make_async_copy(tt_hbm.at[:, pl.ds(0, 128)],
                              slabs[k], isems[k]).wait()

    def transpose_slab(k):
        slab, pbuf = slabs[k], pbufs[k]

        @pl.loop(0, 128, unroll=8)
        def _v(v):
            vs = jnp.full((16,), v, jnp.int32)
            half = (v & 1) * EMB
            for g in range(EMB // 16):
                vals = plsc.load_gather(slab, [_eg[g], vs])
                pbuf[v >> 1, pl.ds(half + g * 16, 16)] = vals

    def fire_out(c, k):
        off = pl.multiple_of(c * 64, 8)
        pltpu.async_copy(pbufs[k].at[:, pl.ds(0, 128)],
                         tp_hbm.at[pl.ds(off, EMB)], osems[k])

    def drain_out(k):
        pltpu.make_async_copy(pbufs[k].at[:, pl.ds(0, 128)],
                              tp_hbm.at[pl.ds(0, EMB)], osems[k]).wait()

    def do_col(c, k, drain_o=True, fire_i=True):
        drain_in(k)
        if drain_o:
            drain_out(k)
        transpose_slab(k)
        if fire_i:
            fire_in(c + 2, k)
        fire_out(c, k)

    fire_in(lo, 0)
    fire_in(lo + 1, 1)
    do_col(lo, 0, drain_o=False)
    do_col(lo + 1, 1, drain_o=False)

    @pl.loop(lo + 2, lo + COLS_BASE - 2, step=2)
    def _cols(c0):
        do_col(c0, 0)
        do_col(c0 + 1, 1)

    do_col(lo + COLS_BASE - 2, 0, fire_i=False)
    do_col(lo + COLS_BASE - 1, 1, fire_i=False)
    drain_out(0)
    drain_out(1)

    @pl.when(extra)
    def _extra_col():
        c = lo + COLS_BASE
        fire_in(c, 0)
        drain_in(0)
        transpose_slab(0)
        pltpu.sync_copy(pbufs[0].at[:, pl.ds(0, 128)],
                        tp_hbm.at[pl.ds(pl.multiple_of(c * 64, 8), EMB)])

    @pl.when(wid == NW - 1)
    def _tail():
        pltpu.sync_copy(tail_hbm, slabs[1])
        transpose_slab(1)
        pltpu.sync_copy(pbufs[1].at[:, pl.ds(0, 128)],
                        tp_hbm.at[pl.ds((VOCAB - 128) // 2, EMB)])


@functools.partial(
    pl.kernel,
    out_type=jax.ShapeDtypeStruct((L, EMB // 8, NW, 8, CHUNK), jnp.float32),
    mesh=_mesh,
    scratch_types=(
        [pltpu.VMEM((L, CHUNK), jnp.int32)]
        + [pltpu.VMEM((CHUNK, EMB), jnp.float32)
           for _ in range(NB)]
        + [pltpu.VMEM((EMB // 8, 8, TPITCH), jnp.float32)
           for _ in range(2)]
        + [pltpu.SemaphoreType.DMA for _ in range(NB + 2)]
    ),
    compiler_params=pltpu.CompilerParams(use_tc_tiling_on_sc=False, needs_layout_passes=False),
)
def _emb_lookup(xt_hbm, table_hbm, out_hbm, idx_v,
                bank0, bank1, bank2, tb0, tb1,
                g0, g1, g2, o0, o1):
    banks = (bank0, bank1, bank2)
    tbufs = (tb0, tb1)
    gsems = (g0, g1, g2)
    osems = (o0, o1)

    wid = lax.axis_index("s") * NUM_CORES + lax.axis_index("c")
    pltpu.sync_copy(xt_hbm.at[:, pl.ds(wid * CHUNK, CHUNK)], idx_v)

    def fire_gather(r, bi):
        pltpu.async_copy(table_hbm.at[idx_v.at[r]], banks[bi], gsems[bi])

    def drain_gather(bi):
        pltpu.make_async_copy(table_hbm.at[pl.ds(0, CHUNK)],
                              banks[bi], gsems[bi]).wait()

    _es = [lax.iota(jnp.int32, 16) + 16 * j for j in range(EMB // 16)]
    _ers = [e >> 3 for e in _es]
    _ris = [e & 7 for e in _es]

    def transpose(bi, ti):
        bank, tbuf = banks[bi], tbufs[ti]

        @pl.loop(0, CHUNK, unroll=8)
        def _rows(b):
            lane = jnp.full((16,), b, jnp.int32)
            for j in range(EMB // 16):
                vals = bank[b, pl.ds(j * 16, 16)]
                plsc.store_scatter(tbuf, [_ers[j], _ris[j], lane], vals)

    def fire_write(r, ti):
        pltpu.async_copy(tbufs[ti].at[:, :, pl.ds(0, CHUNK)],
                         out_hbm.at[r, :, wid], osems[ti])

    def drain_write(ti):
        pltpu.make_async_copy(tbufs[ti].at[:, :, pl.ds(0, CHUNK)],
                              out_hbm.at[0, :, 0], osems[ti]).wait()

    def do_round(r, bi, ti, drain_w=True, fire_g=True):
        if fire_g:
            fire_gather(r + 2, (bi + 2) % NB)
        drain_gather(bi)
        if drain_w:
            drain_write(ti)
        transpose(bi, ti)
        fire_write(r, ti)

    fire_gather(0, 0)
    fire_gather(1, 1)
    do_round(0, 0, 0, drain_w=False)
    do_round(1, 1, 1, drain_w=False)

    @pl.loop(2, L - 6, step=6)
    def _rounds(r0):
        for k in range(6):
            do_round(r0 + k, (2 + k) % NB, k % 2)

    for k in range(6):
        r = L - 6 + k
        do_round(r, r % NB, r % 2, fire_g=(r + 2 < L))
    drain_write(0)
    drain_write(1)


def kernel(x, table):
    tp = _prep_table(table.T, table[VOCAB - 128:].T)
    out = _emb_lookup(x.T.astype(jnp.int32), tp.reshape(VPAD, EMB))
    return out.transpose(2, 4, 0, 1, 3).reshape(B, L, EMB)


# speedup vs baseline: 1.9190x; 1.9190x over previous
import functools

import jax
import jax.numpy as jnp
from jax import lax
from jax.experimental import pallas as pl
from jax.experimental.pallas import tpu as pltpu
from jax.experimental.pallas import tpu_sc as plsc

B = 4096
L = 200
EMB = 64
NUM_CORES = 2
NUM_SUBCORES = 16
NW = NUM_CORES * NUM_SUBCORES
CHUNK = 128
NB = 3
TPITCH = 129

_mesh = plsc.VectorSubcoreMesh(core_axis_name="c", subcore_axis_name="s")


@functools.partial(
    pl.kernel,
    out_type=jax.ShapeDtypeStruct((L, EMB // 8, NW, 8, CHUNK), jnp.float32),
    mesh=_mesh,
    scratch_types=(
        [pltpu.VMEM((L, CHUNK), jnp.int32)]
        + [pltpu.VMEM((CHUNK, EMB), jnp.float32)
           for _ in range(NB)]
        + [pltpu.VMEM((EMB // 8, 8, TPITCH), jnp.float32)
           for _ in range(2)]
        + [pltpu.SemaphoreType.DMA for _ in range(NB + 2)]
    ),
    compiler_params=pltpu.CompilerParams(use_tc_tiling_on_sc=False, needs_layout_passes=False),
)
def _emb_lookup(xt_hbm, table_hbm, out_hbm, idx_v,
                bank0, bank1, bank2, tb0, tb1,
                g0, g1, g2, o0, o1):
    banks = (bank0, bank1, bank2)
    tbufs = (tb0, tb1)
    gsems = (g0, g1, g2)
    osems = (o0, o1)

    wid = lax.axis_index("s") * NUM_CORES + lax.axis_index("c")
    pltpu.sync_copy(xt_hbm.at[:, pl.ds(wid * CHUNK, CHUNK)], idx_v)

    def fire_gather(r, bi):
        pltpu.async_copy(table_hbm.at[idx_v.at[r]], banks[bi], gsems[bi])

    def drain_gather(bi):
        pltpu.make_async_copy(table_hbm.at[pl.ds(0, CHUNK)],
                              banks[bi], gsems[bi]).wait()

    _es = [lax.iota(jnp.int32, 16) + 16 * j for j in range(EMB // 16)]
    _ers = [e >> 3 for e in _es]
    _ris = [e & 7 for e in _es]

    def transpose(bi, ti):
        bank, tbuf = banks[bi], tbufs[ti]

        @pl.loop(0, CHUNK, unroll=8)
        def _rows(b):
            lane = jnp.full((16,), b, jnp.int32)
            for j in range(EMB // 16):
                vals = bank[b, pl.ds(j * 16, 16)]
                plsc.store_scatter(tbuf, [_ers[j], _ris[j], lane], vals)

    def fire_write(r, ti):
        pltpu.async_copy(tbufs[ti].at[:, :, pl.ds(0, CHUNK)],
                         out_hbm.at[r, :, wid], osems[ti])

    def drain_write(ti):
        pltpu.make_async_copy(tbufs[ti].at[:, :, pl.ds(0, CHUNK)],
                              out_hbm.at[0, :, 0], osems[ti]).wait()

    def do_round(r, bi, ti, drain_w=True, fire_g=True):
        if fire_g:
            fire_gather(r + 2, (bi + 2) % NB)
        drain_gather(bi)
        if drain_w:
            drain_write(ti)
        transpose(bi, ti)
        fire_write(r, ti)

    fire_gather(0, 0)
    fire_gather(1, 1)
    do_round(0, 0, 0, drain_w=False)
    do_round(1, 1, 1, drain_w=False)

    @pl.loop(2, L - 6, step=6)
    def _rounds(r0):
        for k in range(6):
            do_round(r0 + k, (2 + k) % NB, k % 2)

    for k in range(6):
        r = L - 6 + k
        do_round(r, r % NB, r % 2, fire_g=(r + 2 < L))
    drain_write(0)
    drain_write(1)


def kernel(x, table):
    out = _emb_lookup(x.T.astype(jnp.int32), table)
    return out.transpose(2, 4, 0, 1, 3).reshape(B, L, EMB)
